# baseline (speedup 1.0000x reference)
import functools

import jax
import jax.numpy as jnp
from jax import lax
from jax.experimental import pallas as pl
from jax.experimental.pallas import tpu as pltpu
from jax.experimental.pallas import tpu_sc as plsc

N = 10000
E = 320000
D_IN = 128
H = 128
T = 32
DE = 16
NH = 2
DH = H // NH

NC = 2
NS = 16
NW = NC * NS
CHUNK = 128
CG = 64
PW = 10240
CPW = PW // CG
FB = 64
E_PAD = NW * PW
E_CH = E_PAD // CHUNK
BK = 1024
NB_E = E_PAD // BK
DT = 256
N_ACC = 10240
R = N_ACC // NW
IBLK = 4096
NIB = E_PAD // IBLK
NG = IBLK // 16
NBLK = 400
NB_N = N // NBLK


def _node_body(x, mem, mem_ts, mail, mail_ts, hh, hist_ts, rem,
               W_t, b_t, Wi_m, Wi_t, Wh, bi, bh, W_feat, b_feat,
               W_ct, b_ct, Wc1_h, Wc1_t, bc1, Wc2, bc2, Wq, Wk_h, Wv_h,
               h_out, dtab_out, stab_out):
    mts = mail_ts[...]
    tf = jnp.cos((mts - mem_ts[...]) * W_t[...] + b_t[...])
    gi = mail[...] @ Wi_m[...] + tf @ Wi_t[...] + bi[...]
    gh = mem[...] @ Wh[...] + bh[...]
    i_r, i_z, i_n = gi[:, :H], gi[:, H:2 * H], gi[:, 2 * H:]
    h_r, h_z, h_n = gh[:, :H], gh[:, H:2 * H], gh[:, 2 * H:]
    r = jax.nn.sigmoid(i_r + h_r)
    z = jax.nn.sigmoid(i_z + h_z)
    n = jnp.tanh(i_n + r * h_n)
    out_mem = (1.0 - z) * n + z * mem[...]
    h0 = out_mem + x[...] @ W_feat[...] + b_feat[...]
    dt = jnp.maximum(mts - hist_ts[...], 0.0)
    te_c = jnp.cos(dt * W_ct[...] + b_ct[...])
    hc = jax.nn.relu(hh[...] @ Wc1_h[...] + te_c @ Wc1_t[...] + bc1[...])
    hc = hc @ Wc2[...] + bc2[...]
    h = jnp.where(rem[...] > 0.5, hc, h0)
    h_out[...] = h
    qh = h @ Wq[...]
    dtab_out[...] = jnp.concatenate(
        [qh, mts, jnp.zeros((NBLK, DT - H - 1), jnp.float32)], axis=1)
    stab_out[...] = jnp.concatenate([h @ Wk_h[...], h @ Wv_h[...]], axis=1)


def _full(shape):
    return pl.BlockSpec(shape, lambda i: (0, 0))


def _node_call(x, memory, mem_ts2, mailbox, mail_ts2, h_hist, hist_ts2, rem,
               W_t, b_t, Wi_m, Wi_t, Wh, bi, bh, W_feat, b_feat,
               W_ct, b_ct, Wc1_h, Wc1_t, bc1, Wc2, bc2, Wq, Wk_h, Wv_h):
    blk = lambda w: pl.BlockSpec((NBLK, w), lambda i: (i, 0))
    args = (x, memory, mem_ts2, mailbox, mail_ts2, h_hist, hist_ts2, rem,
            W_t, b_t, Wi_m, Wi_t, Wh, bi, bh, W_feat, b_feat,
            W_ct, b_ct, Wc1_h, Wc1_t, bc1, Wc2, bc2, Wq, Wk_h, Wv_h)
    in_specs = [blk(D_IN), blk(H), blk(1), blk(H), blk(1), blk(H), blk(1),
                blk(1)] + [_full(a.shape) for a in args[8:]]
    return pl.pallas_call(
        _node_body,
        grid=(NB_N,),
        in_specs=in_specs,
        out_specs=[blk(H), blk(DT), blk(DT)],
        out_shape=[jax.ShapeDtypeStruct((N, H), jnp.float32),
                   jax.ShapeDtypeStruct((N, DT), jnp.float32),
                   jax.ShapeDtypeStruct((N, DT), jnp.float32)],
    )(*args)


def _sc_gather_body(dtab, stab, gdst3, gsrc3, qdt_out, kv_out,
                    dsti_v, srci_v, bufd0, bufd1, bufs0, bufs1,
                    semd0, semd1, sems0, sems1):
    wid = lax.axis_index("s") * NC + lax.axis_index("c")
    pltpu.sync_copy(gdst3.at[wid], dsti_v)
    pltpu.sync_copy(gsrc3.at[wid], srci_v)
    pltpu.async_copy(dtab.at[dsti_v.at[0]], bufd0, semd0)
    pltpu.async_copy(stab.at[srci_v.at[0]], bufs0, sems0)

    def body(j, carry):
        a = 2 * j
        b = a + 1
        pltpu.async_copy(dtab.at[dsti_v.at[b]], bufd1, semd1)
        pltpu.async_copy(stab.at[srci_v.at[b]], bufs1, sems1)
        pltpu.make_async_copy(dtab.at[dsti_v.at[a]], bufd0, semd0).wait()
        pltpu.make_async_copy(stab.at[srci_v.at[a]], bufs0, sems0).wait()
        base_a = wid * PW + a * CG
        pltpu.sync_copy(bufd0, qdt_out.at[pl.ds(base_a, CG)])
        pltpu.sync_copy(bufs0, kv_out.at[pl.ds(base_a, CG)])

        @pl.when(j < CPW // 2 - 1)
        def _next():
            pltpu.async_copy(dtab.at[dsti_v.at[a + 2]], bufd0, semd0)
            pltpu.async_copy(stab.at[srci_v.at[a + 2]], bufs0, sems0)

        pltpu.make_async_copy(dtab.at[dsti_v.at[b]], bufd1, semd1).wait()
        pltpu.make_async_copy(stab.at[srci_v.at[b]], bufs1, sems1).wait()
        pltpu.sync_copy(bufd1, qdt_out.at[pl.ds(base_a + CG, CG)])
        pltpu.sync_copy(bufs1, kv_out.at[pl.ds(base_a + CG, CG)])
        return carry

    lax.fori_loop(0, CPW // 2, body, 0)


def _edge_body(qdt, kv, ets, ef, W_te, b_te, W_edge, contrib):
    blk = qdt[...]
    qd = blk[:, :H]
    td = blk[:, H:H + 1]
    dt = td - ets[...]
    te = jnp.cos(dt * W_te[...] + b_te[...])
    tef = jnp.concatenate([te, ef[...]], axis=1)
    kxvx = tef @ W_edge[...]
    kvb = kv[...]
    k = kvb[:, :H] + kxvx[:, :H]
    v = kvb[:, H:] + kxvx[:, H:]
    qk = qd * k
    s0 = jnp.sum(qk[:, :DH], axis=1, keepdims=True) * 0.125
    s1 = jnp.sum(qk[:, DH:], axis=1, keepdims=True) * 0.125
    e0 = jnp.exp(s0)
    e1 = jnp.exp(s1)
    contrib[...] = jnp.concatenate(
        [e0 * v[:, :DH], e1 * v[:, DH:], e0, e1,
         jnp.zeros((BK, DT - H - 2), jnp.float32)], axis=1)


def _edge_call(qdt, kvt, ets_pad, ef_pad, W_te_r, b_te_r, W_edge):
    blk = lambda w: pl.BlockSpec((BK, w), lambda i: (i, 0))
    return pl.pallas_call(
        _edge_body,
        grid=(NB_E,),
        in_specs=[blk(DT), blk(DT), blk(1), blk(DE),
                  _full(W_te_r.shape), _full(b_te_r.shape), _full(W_edge.shape)],
        out_specs=blk(DT),
        out_shape=jax.ShapeDtypeStruct((E_PAD, DT), jnp.float32),
    )(qdt, kvt, ets_pad, ef_pad, W_te_r, b_te_r, W_edge)


def _sc_scatter_body(contrib, sdst, out, idxb, pos_v, lv_v, bufr0, bufr1,
                     acc, semr0, semr1):
    c = lax.axis_index("c")
    s = lax.axis_index("s")
    w = c * NS + s
    zero16f = jnp.zeros((16,), jnp.float32)
    zero16i = jnp.zeros((16,), jnp.int32)
    iota16 = lax.iota(jnp.int32, 16)
    dn = lax.GatherDimensionNumbers(offset_dims=(), collapsed_slice_dims=(0,),
                                    start_index_map=(0,))

    def tree_add(t):
        for k in (8, 4, 2, 1):
            perm = ((iota16 + k) & 15)[:, None]
            rot = lax.gather(t, perm, dn, slice_sizes=(1,),
                             mode=lax.GatherScatterMode.PROMISE_IN_BOUNDS)
            t = t + rot
        return t[0]

    for k in range(11):
        pos_v[pl.ds(k * 16, 16)] = zero16i
        lv_v[pl.ds(k * 16, 16)] = zero16i

    def zrow(i, carry):
        for k in range(16):
            acc[i, pl.ds(k * 16, 16)] = zero16f
        return carry

    lax.fori_loop(0, R, zrow, 0)

    def accum_batch(nrows):
        cp0 = pltpu.async_copy(contrib.at[pos_v.at[pl.ds(0, FB)]], bufr0, semr0)
        cp1 = pltpu.async_copy(contrib.at[pos_v.at[pl.ds(FB, FB)]], bufr1, semr1)
        cp0.wait()

        def acc_row0(r, carry):
            @pl.when(r < nrows)
            def _do():
                l = lv_v[pl.ds(r, 16)][0]
                for k in range(16):
                    plsc.addupdate(acc.at[l, pl.ds(k * 16, 16)],
                                   bufr0[r, pl.ds(k * 16, 16)])

            return carry

        lax.fori_loop(0, FB, acc_row0, 0)
        cp1.wait()

        def acc_row1(r, carry):
            @pl.when(r + FB < nrows)
            def _do():
                l = lv_v[pl.ds(r + FB, 16)][0]
                for k in range(16):
                    plsc.addupdate(acc.at[l, pl.ds(k * 16, 16)],
                                   bufr1[r, pl.ds(k * 16, 16)])

            return carry

        lax.fori_loop(0, FB, acc_row1, 0)

    def flush_check(cc):
        @pl.when(cc >= CHUNK)
        def _flush():
            accum_batch(jnp.int32(CHUNK))
            p1 = pos_v[pl.ds(CHUNK, 16)]
            l1 = lv_v[pl.ds(CHUNK, 16)]
            p2 = pos_v[pl.ds(CHUNK + 16, 16)]
            l2 = lv_v[pl.ds(CHUNK + 16, 16)]
            pos_v[pl.ds(0, 16)] = p1
            lv_v[pl.ds(0, 16)] = l1
            pos_v[pl.ds(16, 16)] = p2
            lv_v[pl.ds(16, 16)] = l2

        return jnp.where(cc >= CHUNK, cc - CHUNK, cc)

    def blk(b, cnt):
        pltpu.sync_copy(sdst.at[pl.ds(b * IBLK, IBLK)], idxb)

        def sgrp(q, cnt2):
            goff = q * CHUNK
            base128 = b * IBLK + goff
            lvs = []
            ms = []
            for g in range(8):
                iv = idxb[pl.ds(goff + g * 16, 16)]
                lv = iv - w * R
                m = (lv >= 0) & (lv < R)
                lvs.append(lv)
                ms.append(m)
            acc0 = jnp.where(ms[0], 1, 0)
            acc1 = jnp.where(ms[4], 1, 0)
            for g in range(1, 4):
                acc0 = acc0 + jnp.where(ms[g], 1 << (8 * g), 0)
                acc1 = acc1 + jnp.where(ms[4 + g], 1 << (8 * g), 0)
            bits0 = tree_add(acc0)
            bits1 = tree_add(acc1)

            cc = cnt2
            for g in range(8):
                bits = bits0 if g < 4 else bits1
                cnt_g = (bits >> (8 * (g % 4))) & 255
                base_g = base128 + g * 16
                lv_g = lvs[g]
                m_g = ms[g]

                @pl.when(cnt_g == 1)
                def _one(lv_g=lv_g, m_g=m_g, base_g=base_g, cc=cc):
                    enc = tree_add(jnp.where(m_g, lv_g * 16 + iota16, 0))
                    lane = enc & 15
                    lval = enc >> 4
                    pos_v[pl.ds(cc, 16)] = jnp.full((16,), base_g + lane,
                                                    jnp.int32)
                    lv_v[pl.ds(cc, 16)] = jnp.full((16,), lval, jnp.int32)

                @pl.when(cnt_g > 1)
                def _multi(lv_g=lv_g, base_g=base_g, cc=cc):
                    ccl = cc
                    for lane in range(16):
                        lvl = lv_g[lane]
                        cond = (lvl >= 0) & (lvl < R)

                        @pl.when(cond)
                        def _st(lvl=lvl, ccl=ccl, lane=lane):
                            pos_v[pl.ds(ccl, 16)] = jnp.full(
                                (16,), base_g + lane, jnp.int32)
                            lv_v[pl.ds(ccl, 16)] = jnp.full((16,), lvl,
                                                            jnp.int32)

                        ccl = ccl + jnp.where(cond, 1, 0)

                cc = cc + cnt_g
                cc = flush_check(cc)
            return cc

        return lax.fori_loop(0, IBLK // CHUNK, sgrp, cnt)

    cnt_end = lax.fori_loop(0, NIB, blk, jnp.int32(0))

    @pl.when(cnt_end > 0)
    def _tail():
        accum_batch(cnt_end)

    pltpu.sync_copy(acc, out.at[pl.ds(w * R, R)])


@functools.lru_cache(maxsize=None)
def _build_sc_kernels():
    mesh = plsc.VectorSubcoreMesh(core_axis_name="c", subcore_axis_name="s",
                                  num_cores=NC, num_subcores=NS)
    gather = pl.kernel(
        _sc_gather_body,
        out_type=(jax.ShapeDtypeStruct((E_PAD, DT), jnp.float32),
                  jax.ShapeDtypeStruct((E_PAD, DT), jnp.float32)),
        mesh=mesh,
        scratch_types=[pltpu.VMEM((CPW, CG), jnp.int32),
                       pltpu.VMEM((CPW, CG), jnp.int32),
                       pltpu.VMEM((CG, DT), jnp.float32),
                       pltpu.VMEM((CG, DT), jnp.float32),
                       pltpu.VMEM((CG, DT), jnp.float32),
                       pltpu.VMEM((CG, DT), jnp.float32),
                       pltpu.SemaphoreType.DMA,
                       pltpu.SemaphoreType.DMA,
                       pltpu.SemaphoreType.DMA,
                       pltpu.SemaphoreType.DMA])
    scatter = pl.kernel(
        _sc_scatter_body,
        out_type=jax.ShapeDtypeStruct((N_ACC, DT), jnp.float32),
        mesh=mesh,
        scratch_types=[pltpu.VMEM((IBLK,), jnp.int32),
                       pltpu.VMEM((176,), jnp.int32),
                       pltpu.VMEM((176,), jnp.int32),
                       pltpu.VMEM((FB, DT), jnp.float32),
                       pltpu.VMEM((FB, DT), jnp.float32),
                       pltpu.VMEM((R, DT), jnp.float32),
                       pltpu.SemaphoreType.DMA,
                       pltpu.SemaphoreType.DMA])
    return gather, scatter


def _sc_gather(dtab, stab, gdst3, gsrc3):
    return _build_sc_kernels()[0](dtab, stab, gdst3, gsrc3)


def _sc_scatter(contrib, sdst):
    return _build_sc_kernels()[1](contrib, sdst)


def _final_body(p0, h, Wo_a, Wo_h, bo, out):
    a = p0[...]
    den0 = a[:, H:H + 1] + 1e-16
    den1 = a[:, H + 1:H + 2] + 1e-16
    agg = jnp.concatenate([a[:, :DH] / den0, a[:, DH:H] / den1], axis=1)
    out[...] = agg @ Wo_a[...] + h[...] @ Wo_h[...] + bo[...]


def _final_call(p0, h, Wo_a, Wo_h, bo_r):
    blk = lambda w: pl.BlockSpec((NBLK, w), lambda i: (i, 0))
    return pl.pallas_call(
        _final_body,
        grid=(NB_N,),
        in_specs=[blk(DT), blk(H),
                  _full(Wo_a.shape), _full(Wo_h.shape), _full(bo_r.shape)],
        out_specs=blk(H),
        out_shape=jax.ShapeDtypeStruct((N, H), jnp.float32),
    )(p0, h, Wo_a, Wo_h, bo_r)


def kernel(x, memory, mem_ts, mailbox, mail_ts, edge_ts, edge_feat, h_hist,
           hist_ts, W_t, b_t, W_te, b_te, Wi, Wh, bi, bh, W_feat, b_feat,
           Wq, Wk, Wv, Wo, bo, W_ct, b_ct, Wc1, bc1, Wc2, bc2,
           edge_index, is_remote):
    f32 = jnp.float32
    row = lambda w: w[None, :]
    h, dtab, stab = _node_call(
        x, memory, mem_ts[:, None], mailbox, mail_ts[:, None], h_hist,
        hist_ts[:, None], is_remote[:, None].astype(f32),
        row(W_t), row(b_t), Wi[:H], Wi[H:], Wh, row(bi), row(bh),
        W_feat, row(b_feat), row(W_ct), row(b_ct), Wc1[:H], Wc1[H:],
        row(bc1), Wc2, row(bc2), Wq, Wk[:H], Wv[:H])

    src = edge_index[0]
    dst = edge_index[1]
    pad = E_PAD - E
    gdst3 = jnp.pad(dst, (0, pad)).reshape(NW, CPW, CG)
    gsrc3 = jnp.pad(src, (0, pad)).reshape(NW, CPW, CG)
    sdst = jnp.pad(dst, (0, pad), constant_values=N)
    ets_pad = jnp.pad(edge_ts, (0, pad))[:, None]
    ef_pad = jnp.pad(edge_feat, ((0, pad), (0, 0)))

    qdt, kvt = _sc_gather(dtab, stab, gdst3, gsrc3)
    W_edge = jnp.concatenate([Wk[H:], Wv[H:]], axis=1)
    contrib = _edge_call(qdt, kvt, ets_pad, ef_pad, row(W_te), row(b_te), W_edge)
    acc = _sc_scatter(contrib, sdst)
    return _final_call(acc, h, Wo[:H], Wo[H:], row(bo))


# speedup vs baseline: 1.3720x; 1.3720x over previous
import functools

import jax
import jax.numpy as jnp
from jax import lax
from jax.experimental import pallas as pl
from jax.experimental.pallas import tpu as pltpu
from jax.experimental.pallas import tpu_sc as plsc

N = 10000
E = 320000
D_IN = 128
H = 128
T = 32
DE = 16
NH = 2
DH = H // NH

NC = 2
NS = 16
NW = NC * NS
CHUNK = 128
CG = 64
PW = 10240
CPW = PW // CG
FB = 64
E_PAD = NW * PW
E_CH = E_PAD // CHUNK
BK = 1024
NB_E = E_PAD // BK
DT = 256
N_ACC = 10240
R = N_ACC // NW
IBLK = 4096
NIB = E_PAD // IBLK
NG = IBLK // 16
NBLK = 400
NB_N = N // NBLK


def _node_body(x, mem, mem_ts, mail, mail_ts, hh, hist_ts, rem,
               W_t, b_t, Wi_m, Wi_t, Wh, bi, bh, W_feat, b_feat,
               W_ct, b_ct, Wc1_h, Wc1_t, bc1, Wc2, bc2, Wq, Wk_h, Wv_h,
               h_out, dtab_out, stab_out):
    mts = mail_ts[...]
    tf = jnp.cos((mts - mem_ts[...]) * W_t[...] + b_t[...])
    gi = mail[...] @ Wi_m[...] + tf @ Wi_t[...] + bi[...]
    gh = mem[...] @ Wh[...] + bh[...]
    i_r, i_z, i_n = gi[:, :H], gi[:, H:2 * H], gi[:, 2 * H:]
    h_r, h_z, h_n = gh[:, :H], gh[:, H:2 * H], gh[:, 2 * H:]
    r = jax.nn.sigmoid(i_r + h_r)
    z = jax.nn.sigmoid(i_z + h_z)
    n = jnp.tanh(i_n + r * h_n)
    out_mem = (1.0 - z) * n + z * mem[...]
    h0 = out_mem + x[...] @ W_feat[...] + b_feat[...]
    dt = jnp.maximum(mts - hist_ts[...], 0.0)
    te_c = jnp.cos(dt * W_ct[...] + b_ct[...])
    hc = jax.nn.relu(hh[...] @ Wc1_h[...] + te_c @ Wc1_t[...] + bc1[...])
    hc = hc @ Wc2[...] + bc2[...]
    h = jnp.where(rem[...] > 0.5, hc, h0)
    h_out[...] = h
    qh = h @ Wq[...]
    dtab_out[...] = jnp.concatenate(
        [qh, mts, jnp.zeros((NBLK, DT - H - 1), jnp.float32)], axis=1)
    stab_out[...] = jnp.concatenate([h @ Wk_h[...], h @ Wv_h[...]], axis=1)


def _full(shape):
    return pl.BlockSpec(shape, lambda i: (0, 0))


def _node_call(x, memory, mem_ts2, mailbox, mail_ts2, h_hist, hist_ts2, rem,
               W_t, b_t, Wi_m, Wi_t, Wh, bi, bh, W_feat, b_feat,
               W_ct, b_ct, Wc1_h, Wc1_t, bc1, Wc2, bc2, Wq, Wk_h, Wv_h):
    blk = lambda w: pl.BlockSpec((NBLK, w), lambda i: (i, 0))
    args = (x, memory, mem_ts2, mailbox, mail_ts2, h_hist, hist_ts2, rem,
            W_t, b_t, Wi_m, Wi_t, Wh, bi, bh, W_feat, b_feat,
            W_ct, b_ct, Wc1_h, Wc1_t, bc1, Wc2, bc2, Wq, Wk_h, Wv_h)
    in_specs = [blk(D_IN), blk(H), blk(1), blk(H), blk(1), blk(H), blk(1),
                blk(1)] + [_full(a.shape) for a in args[8:]]
    return pl.pallas_call(
        _node_body,
        grid=(NB_N,),
        in_specs=in_specs,
        out_specs=[blk(H), blk(DT), blk(DT)],
        out_shape=[jax.ShapeDtypeStruct((N, H), jnp.float32),
                   jax.ShapeDtypeStruct((N, DT), jnp.float32),
                   jax.ShapeDtypeStruct((N, DT), jnp.float32)],
    )(*args)


def _sc_gather_body(dtab, stab, gdst3, gsrc3, qdt_out, kv_out,
                    dsti_v, srci_v, bufd0, bufd1, bufs0, bufs1,
                    semd0, semd1, sems0, sems1):
    wid = lax.axis_index("s") * NC + lax.axis_index("c")
    pltpu.sync_copy(gdst3.at[wid], dsti_v)
    pltpu.sync_copy(gsrc3.at[wid], srci_v)
    pltpu.async_copy(dtab.at[dsti_v.at[0]], bufd0, semd0)
    pltpu.async_copy(stab.at[srci_v.at[0]], bufs0, sems0)

    def body(j, carry):
        a = 2 * j
        b = a + 1
        pltpu.async_copy(dtab.at[dsti_v.at[b]], bufd1, semd1)
        pltpu.async_copy(stab.at[srci_v.at[b]], bufs1, sems1)
        pltpu.make_async_copy(dtab.at[dsti_v.at[a]], bufd0, semd0).wait()
        pltpu.make_async_copy(stab.at[srci_v.at[a]], bufs0, sems0).wait()
        base_a = wid * PW + a * CG
        pltpu.sync_copy(bufd0, qdt_out.at[pl.ds(base_a, CG)])
        pltpu.sync_copy(bufs0, kv_out.at[pl.ds(base_a, CG)])

        @pl.when(j < CPW // 2 - 1)
        def _next():
            pltpu.async_copy(dtab.at[dsti_v.at[a + 2]], bufd0, semd0)
            pltpu.async_copy(stab.at[srci_v.at[a + 2]], bufs0, sems0)

        pltpu.make_async_copy(dtab.at[dsti_v.at[b]], bufd1, semd1).wait()
        pltpu.make_async_copy(stab.at[srci_v.at[b]], bufs1, sems1).wait()
        pltpu.sync_copy(bufd1, qdt_out.at[pl.ds(base_a + CG, CG)])
        pltpu.sync_copy(bufs1, kv_out.at[pl.ds(base_a + CG, CG)])
        return carry

    lax.fori_loop(0, CPW // 2, body, 0)


def _edge_body(qdt, kv, ets, ef, W_te, b_te, W_edge, contrib):
    blk = qdt[...]
    qd = blk[:, :H]
    td = blk[:, H:H + 1]
    dt = td - ets[...]
    te = jnp.cos(dt * W_te[...] + b_te[...])
    tef = jnp.concatenate([te, ef[...]], axis=1)
    kxvx = tef @ W_edge[...]
    kvb = kv[...]
    k = kvb[:, :H] + kxvx[:, :H]
    v = kvb[:, H:] + kxvx[:, H:]
    qk = qd * k
    s0 = jnp.sum(qk[:, :DH], axis=1, keepdims=True) * 0.125
    s1 = jnp.sum(qk[:, DH:], axis=1, keepdims=True) * 0.125
    e0 = jnp.exp(s0)
    e1 = jnp.exp(s1)
    contrib[...] = jnp.concatenate(
        [e0 * v[:, :DH], e1 * v[:, DH:], e0, e1,
         jnp.zeros((BK, DT - H - 2), jnp.float32)], axis=1)


def _edge_call(qdt, kvt, ets_pad, ef_pad, W_te_r, b_te_r, W_edge):
    blk = lambda w: pl.BlockSpec((BK, w), lambda i: (i, 0))
    return pl.pallas_call(
        _edge_body,
        grid=(NB_E,),
        in_specs=[blk(DT), blk(DT), blk(1), blk(DE),
                  _full(W_te_r.shape), _full(b_te_r.shape), _full(W_edge.shape)],
        out_specs=blk(DT),
        out_shape=jax.ShapeDtypeStruct((E_PAD, DT), jnp.float32),
    )(qdt, kvt, ets_pad, ef_pad, W_te_r, b_te_r, W_edge)


def _sc_scatter_body(contrib, sdst, out, idxb, pos_v, lv_v, bufr0, bufr1,
                     acc, semr0, semr1):
    c = lax.axis_index("c")
    s = lax.axis_index("s")
    w = c * NS + s
    zero16f = jnp.zeros((16,), jnp.float32)
    zero16i = jnp.zeros((16,), jnp.int32)
    iota16 = lax.iota(jnp.int32, 16)
    dn = lax.GatherDimensionNumbers(offset_dims=(), collapsed_slice_dims=(0,),
                                    start_index_map=(0,))

    def tree_add(t):
        for k in (8, 4, 2, 1):
            perm = ((iota16 + k) & 15)[:, None]
            rot = lax.gather(t, perm, dn, slice_sizes=(1,),
                             mode=lax.GatherScatterMode.PROMISE_IN_BOUNDS)
            t = t + rot
        return t[0]

    for k in range(11):
        pos_v[pl.ds(k * 16, 16)] = zero16i
        lv_v[pl.ds(k * 16, 16)] = zero16i

    def zrow(i, carry):
        for k in range(16):
            acc[i, pl.ds(k * 16, 16)] = zero16f
        return carry

    lax.fori_loop(0, R, zrow, 0)

    def accum_batch(nrows):
        cp0 = pltpu.async_copy(contrib.at[pos_v.at[pl.ds(0, FB)]], bufr0, semr0)
        cp1 = pltpu.async_copy(contrib.at[pos_v.at[pl.ds(FB, FB)]], bufr1, semr1)
        cp0.wait()

        def acc_row0(r, carry):
            @pl.when(r < nrows)
            def _do():
                l = lv_v[pl.ds(r, 16)][0]
                for k in range(16):
                    plsc.addupdate(acc.at[l, pl.ds(k * 16, 16)],
                                   bufr0[r, pl.ds(k * 16, 16)])

            return carry

        lax.fori_loop(0, FB, acc_row0, 0)
        cp1.wait()

        def acc_row1(r, carry):
            @pl.when(r + FB < nrows)
            def _do():
                l = lv_v[pl.ds(r + FB, 16)][0]
                for k in range(16):
                    plsc.addupdate(acc.at[l, pl.ds(k * 16, 16)],
                                   bufr1[r, pl.ds(k * 16, 16)])

            return carry

        lax.fori_loop(0, FB, acc_row1, 0)

    def flush_check(cc):
        @pl.when(cc >= CHUNK)
        def _flush():
            accum_batch(jnp.int32(CHUNK))
            p1 = pos_v[pl.ds(CHUNK, 16)]
            l1 = lv_v[pl.ds(CHUNK, 16)]
            p2 = pos_v[pl.ds(CHUNK + 16, 16)]
            l2 = lv_v[pl.ds(CHUNK + 16, 16)]
            pos_v[pl.ds(0, 16)] = p1
            lv_v[pl.ds(0, 16)] = l1
            pos_v[pl.ds(16, 16)] = p2
            lv_v[pl.ds(16, 16)] = l2

        return jnp.where(cc >= CHUNK, cc - CHUNK, cc)

    def blk(b, cnt):
        pltpu.sync_copy(sdst.at[pl.ds(b * IBLK, IBLK)], idxb)

        def grp(g, cnt2):
            iv = idxb[pl.ds(g * 16, 16)]
            lv = iv - w * R
            m = (lv >= 0) & (lv < R)
            base = b * IBLK + g * 16
            packed = tree_add(jnp.where(m, 65536 + lv * 16 + iota16, 0))
            nm = packed >> 16

            @pl.when(nm == 1)
            def _one():
                enc = packed & 65535
                lane = enc & 15
                lval = enc >> 4
                pos_v[pl.ds(cnt2, 16)] = jnp.full((16,), base + lane, jnp.int32)
                lv_v[pl.ds(cnt2, 16)] = jnp.full((16,), lval, jnp.int32)

            @pl.when(nm > 1)
            def _multi():
                cc = cnt2
                for lane in range(16):
                    lvl = lv[lane]
                    cond = (lvl >= 0) & (lvl < R)

                    @pl.when(cond)
                    def _st(lvl=lvl, cc=cc, lane=lane):
                        pos_v[pl.ds(cc, 16)] = jnp.full((16,), base + lane,
                                                        jnp.int32)
                        lv_v[pl.ds(cc, 16)] = jnp.full((16,), lvl, jnp.int32)

                    cc = cc + jnp.where(cond, 1, 0)

            return flush_check(cnt2 + nm)

        return lax.fori_loop(0, NG, grp, cnt)

    cnt_end = lax.fori_loop(0, NIB, blk, jnp.int32(0))

    @pl.when(cnt_end > 0)
    def _tail():
        accum_batch(cnt_end)

    pltpu.sync_copy(acc, out.at[pl.ds(w * R, R)])


@functools.lru_cache(maxsize=None)
def _build_sc_kernels():
    mesh = plsc.VectorSubcoreMesh(core_axis_name="c", subcore_axis_name="s",
                                  num_cores=NC, num_subcores=NS)
    gather = pl.kernel(
        _sc_gather_body,
        out_type=(jax.ShapeDtypeStruct((E_PAD, DT), jnp.float32),
                  jax.ShapeDtypeStruct((E_PAD, DT), jnp.float32)),
        mesh=mesh,
        scratch_types=[pltpu.VMEM((CPW, CG), jnp.int32),
                       pltpu.VMEM((CPW, CG), jnp.int32),
                       pltpu.VMEM((CG, DT), jnp.float32),
                       pltpu.VMEM((CG, DT), jnp.float32),
                       pltpu.VMEM((CG, DT), jnp.float32),
                       pltpu.VMEM((CG, DT), jnp.float32),
                       pltpu.SemaphoreType.DMA,
                       pltpu.SemaphoreType.DMA,
                       pltpu.SemaphoreType.DMA,
                       pltpu.SemaphoreType.DMA])
    scatter = pl.kernel(
        _sc_scatter_body,
        out_type=jax.ShapeDtypeStruct((N_ACC, DT), jnp.float32),
        mesh=mesh,
        scratch_types=[pltpu.VMEM((IBLK,), jnp.int32),
                       pltpu.VMEM((176,), jnp.int32),
                       pltpu.VMEM((176,), jnp.int32),
                       pltpu.VMEM((FB, DT), jnp.float32),
                       pltpu.VMEM((FB, DT), jnp.float32),
                       pltpu.VMEM((R, DT), jnp.float32),
                       pltpu.SemaphoreType.DMA,
                       pltpu.SemaphoreType.DMA])
    return gather, scatter


def _sc_gather(dtab, stab, gdst3, gsrc3):
    return _build_sc_kernels()[0](dtab, stab, gdst3, gsrc3)


def _sc_scatter(contrib, sdst):
    return _build_sc_kernels()[1](contrib, sdst)


def _final_body(p0, h, Wo_a, Wo_h, bo, out):
    a = p0[...]
    den0 = a[:, H:H + 1] + 1e-16
    den1 = a[:, H + 1:H + 2] + 1e-16
    agg = jnp.concatenate([a[:, :DH] / den0, a[:, DH:H] / den1], axis=1)
    out[...] = agg @ Wo_a[...] + h[...] @ Wo_h[...] + bo[...]


def _final_call(p0, h, Wo_a, Wo_h, bo_r):
    blk = lambda w: pl.BlockSpec((NBLK, w), lambda i: (i, 0))
    return pl.pallas_call(
        _final_body,
        grid=(NB_N,),
        in_specs=[blk(DT), blk(H),
                  _full(Wo_a.shape), _full(Wo_h.shape), _full(bo_r.shape)],
        out_specs=blk(H),
        out_shape=jax.ShapeDtypeStruct((N, H), jnp.float32),
    )(p0, h, Wo_a, Wo_h, bo_r)


def kernel(x, memory, mem_ts, mailbox, mail_ts, edge_ts, edge_feat, h_hist,
           hist_ts, W_t, b_t, W_te, b_te, Wi, Wh, bi, bh, W_feat, b_feat,
           Wq, Wk, Wv, Wo, bo, W_ct, b_ct, Wc1, bc1, Wc2, bc2,
           edge_index, is_remote):
    f32 = jnp.float32
    row = lambda w: w[None, :]
    h, dtab, stab = _node_call(
        x, memory, mem_ts[:, None], mailbox, mail_ts[:, None], h_hist,
        hist_ts[:, None], is_remote[:, None].astype(f32),
        row(W_t), row(b_t), Wi[:H], Wi[H:], Wh, row(bi), row(bh),
        W_feat, row(b_feat), row(W_ct), row(b_ct), Wc1[:H], Wc1[H:],
        row(bc1), Wc2, row(bc2), Wq, Wk[:H], Wv[:H])

    src = edge_index[0]
    dst = edge_index[1]
    pad = E_PAD - E
    gdst3 = jnp.pad(dst, (0, pad)).reshape(NW, CPW, CG)
    gsrc3 = jnp.pad(src, (0, pad)).reshape(NW, CPW, CG)
    sdst = jnp.pad(dst, (0, pad), constant_values=N)
    ets_pad = jnp.pad(edge_ts, (0, pad))[:, None]
    ef_pad = jnp.pad(edge_feat, ((0, pad), (0, 0)))

    qdt, kvt = _sc_gather(dtab, stab, gdst3, gsrc3)
    W_edge = jnp.concatenate([Wk[H:], Wv[H:]], axis=1)
    contrib = _edge_call(qdt, kvt, ets_pad, ef_pad, row(W_te), row(b_te), W_edge)
    acc = _sc_scatter(contrib, sdst)
    return _final_call(acc, h, Wo[:H], Wo[H:], row(bo))


# speedup vs baseline: 1.5993x; 1.1657x over previous
import functools

import jax
import jax.numpy as jnp
from jax import lax
from jax.experimental import pallas as pl
from jax.experimental.pallas import tpu as pltpu
from jax.experimental.pallas import tpu_sc as plsc

N = 10000
E = 320000
D_IN = 128
H = 128
T = 32
DE = 16
NH = 2
DH = H // NH

NC = 2
NS = 16
NW = NC * NS
CHUNK = 128
CG = 64
PW = 10240
CPW = PW // CG
FB = 64
E_PAD = NW * PW
E_CH = E_PAD // CHUNK
BK = 1024
NB_E = E_PAD // BK
DT = 256
N_ACC = 10240
R = N_ACC // NW
IBLK = 4096
NIB = E_PAD // IBLK
NG = IBLK // 16
NBLK = 400
NB_N = N // NBLK


def _node_body(x, mem, mem_ts, mail, mail_ts, hh, hist_ts, rem,
               W_t, b_t, Wi_m, Wi_t, Wh, bi, bh, W_feat, b_feat,
               W_ct, b_ct, Wc1_h, Wc1_t, bc1, Wc2, bc2, Wq, Wk_h, Wv_h,
               h_out, dtab_out, stab_out):
    mts = mail_ts[...]
    tf = jnp.cos((mts - mem_ts[...]) * W_t[...] + b_t[...])
    gi = mail[...] @ Wi_m[...] + tf @ Wi_t[...] + bi[...]
    gh = mem[...] @ Wh[...] + bh[...]
    i_r, i_z, i_n = gi[:, :H], gi[:, H:2 * H], gi[:, 2 * H:]
    h_r, h_z, h_n = gh[:, :H], gh[:, H:2 * H], gh[:, 2 * H:]
    r = jax.nn.sigmoid(i_r + h_r)
    z = jax.nn.sigmoid(i_z + h_z)
    n = jnp.tanh(i_n + r * h_n)
    out_mem = (1.0 - z) * n + z * mem[...]
    h0 = out_mem + x[...] @ W_feat[...] + b_feat[...]
    dt = jnp.maximum(mts - hist_ts[...], 0.0)
    te_c = jnp.cos(dt * W_ct[...] + b_ct[...])
    hc = jax.nn.relu(hh[...] @ Wc1_h[...] + te_c @ Wc1_t[...] + bc1[...])
    hc = hc @ Wc2[...] + bc2[...]
    h = jnp.where(rem[...] > 0.5, hc, h0)
    h_out[...] = h
    qh = h @ Wq[...]
    dtab_out[...] = jnp.concatenate(
        [qh, mts, jnp.zeros((NBLK, DT - H - 1), jnp.float32)], axis=1)
    stab_out[...] = jnp.concatenate([h @ Wk_h[...], h @ Wv_h[...]], axis=1)


def _full(shape):
    return pl.BlockSpec(shape, lambda i: (0, 0))


def _node_call(x, memory, mem_ts2, mailbox, mail_ts2, h_hist, hist_ts2, rem,
               W_t, b_t, Wi_m, Wi_t, Wh, bi, bh, W_feat, b_feat,
               W_ct, b_ct, Wc1_h, Wc1_t, bc1, Wc2, bc2, Wq, Wk_h, Wv_h):
    blk = lambda w: pl.BlockSpec((NBLK, w), lambda i: (i, 0))
    args = (x, memory, mem_ts2, mailbox, mail_ts2, h_hist, hist_ts2, rem,
            W_t, b_t, Wi_m, Wi_t, Wh, bi, bh, W_feat, b_feat,
            W_ct, b_ct, Wc1_h, Wc1_t, bc1, Wc2, bc2, Wq, Wk_h, Wv_h)
    in_specs = [blk(D_IN), blk(H), blk(1), blk(H), blk(1), blk(H), blk(1),
                blk(1)] + [_full(a.shape) for a in args[8:]]
    return pl.pallas_call(
        _node_body,
        grid=(NB_N,),
        in_specs=in_specs,
        out_specs=[blk(H), blk(DT), blk(DT)],
        out_shape=[jax.ShapeDtypeStruct((N, H), jnp.float32),
                   jax.ShapeDtypeStruct((N, DT), jnp.float32),
                   jax.ShapeDtypeStruct((N, DT), jnp.float32)],
    )(*args)


def _sc_gather_body(dtab, stab, gdst3, gsrc3, qdt_out, kv_out,
                    dsti_v, srci_v, bufd0, bufd1, bufs0, bufs1,
                    semd0, semd1, sems0, sems1):
    wid = lax.axis_index("s") * NC + lax.axis_index("c")
    pltpu.sync_copy(gdst3.at[wid], dsti_v)
    pltpu.sync_copy(gsrc3.at[wid], srci_v)
    pltpu.async_copy(dtab.at[dsti_v.at[0]], bufd0, semd0)
    pltpu.async_copy(stab.at[srci_v.at[0]], bufs0, sems0)

    def body(j, carry):
        a = 2 * j
        b = a + 1
        pltpu.async_copy(dtab.at[dsti_v.at[b]], bufd1, semd1)
        pltpu.async_copy(stab.at[srci_v.at[b]], bufs1, sems1)
        pltpu.make_async_copy(dtab.at[dsti_v.at[a]], bufd0, semd0).wait()
        pltpu.make_async_copy(stab.at[srci_v.at[a]], bufs0, sems0).wait()
        base_a = wid * PW + a * CG
        pltpu.sync_copy(bufd0, qdt_out.at[pl.ds(base_a, CG)])
        pltpu.sync_copy(bufs0, kv_out.at[pl.ds(base_a, CG)])

        @pl.when(j < CPW // 2 - 1)
        def _next():
            pltpu.async_copy(dtab.at[dsti_v.at[a + 2]], bufd0, semd0)
            pltpu.async_copy(stab.at[srci_v.at[a + 2]], bufs0, sems0)

        pltpu.make_async_copy(dtab.at[dsti_v.at[b]], bufd1, semd1).wait()
        pltpu.make_async_copy(stab.at[srci_v.at[b]], bufs1, sems1).wait()
        pltpu.sync_copy(bufd1, qdt_out.at[pl.ds(base_a + CG, CG)])
        pltpu.sync_copy(bufs1, kv_out.at[pl.ds(base_a + CG, CG)])
        return carry

    lax.fori_loop(0, CPW // 2, body, 0)


def _edge_body(qdt, kv, ets, ef, W_te, b_te, W_edge, contrib):
    blk = qdt[...]
    qd = blk[:, :H]
    td = blk[:, H:H + 1]
    dt = td - ets[...]
    te = jnp.cos(dt * W_te[...] + b_te[...])
    tef = jnp.concatenate([te, ef[...]], axis=1)
    kxvx = tef @ W_edge[...]
    kvb = kv[...]
    k = kvb[:, :H] + kxvx[:, :H]
    v = kvb[:, H:] + kxvx[:, H:]
    qk = qd * k
    s0 = jnp.sum(qk[:, :DH], axis=1, keepdims=True) * 0.125
    s1 = jnp.sum(qk[:, DH:], axis=1, keepdims=True) * 0.125
    e0 = jnp.exp(s0)
    e1 = jnp.exp(s1)
    contrib[...] = jnp.concatenate(
        [e0 * v[:, :DH], e1 * v[:, DH:], e0, e1,
         jnp.zeros((BK, DT - H - 2), jnp.float32)], axis=1)


def _edge_call(qdt, kvt, ets_pad, ef_pad, W_te_r, b_te_r, W_edge):
    blk = lambda w: pl.BlockSpec((BK, w), lambda i: (i, 0))
    return pl.pallas_call(
        _edge_body,
        grid=(NB_E,),
        in_specs=[blk(DT), blk(DT), blk(1), blk(DE),
                  _full(W_te_r.shape), _full(b_te_r.shape), _full(W_edge.shape)],
        out_specs=blk(DT),
        out_shape=jax.ShapeDtypeStruct((E_PAD, DT), jnp.float32),
    )(qdt, kvt, ets_pad, ef_pad, W_te_r, b_te_r, W_edge)


def _sc_scatter_body(contrib, sdst, out, idxb, pos_v, lv_v, lvtmp, bufr0,
                     bufr1, acc, semr0, semr1):
    c = lax.axis_index("c")
    s = lax.axis_index("s")
    w = c * NS + s
    zero16f = jnp.zeros((16,), jnp.float32)
    zero16i = jnp.zeros((16,), jnp.int32)
    iota16 = lax.iota(jnp.int32, 16)
    dn = lax.GatherDimensionNumbers(offset_dims=(), collapsed_slice_dims=(0,),
                                    start_index_map=(0,))

    def tree_add(t):
        for k in (8, 4, 2, 1):
            perm = ((iota16 + k) & 15)[:, None]
            rot = lax.gather(t, perm, dn, slice_sizes=(1,),
                             mode=lax.GatherScatterMode.PROMISE_IN_BOUNDS)
            t = t + rot
        return t[0]

    for k in range(11):
        pos_v[pl.ds(k * 16, 16)] = zero16i
        lv_v[pl.ds(k * 16, 16)] = zero16i

    def zrow(i, carry):
        for k in range(16):
            acc[i, pl.ds(k * 16, 16)] = zero16f
        return carry

    lax.fori_loop(0, R, zrow, 0)

    def accum_batch(nrows):
        cp0 = pltpu.async_copy(contrib.at[pos_v.at[pl.ds(0, FB)]], bufr0, semr0)
        cp1 = pltpu.async_copy(contrib.at[pos_v.at[pl.ds(FB, FB)]], bufr1, semr1)
        cp0.wait()

        def acc_row0(r, carry):
            @pl.when(r < nrows)
            def _do():
                l = lv_v[pl.ds(r, 16)][0]
                for k in range(16):
                    plsc.addupdate(acc.at[l, pl.ds(k * 16, 16)],
                                   bufr0[r, pl.ds(k * 16, 16)])

            return carry

        lax.fori_loop(0, FB, acc_row0, 0)
        cp1.wait()

        def acc_row1(r, carry):
            @pl.when(r + FB < nrows)
            def _do():
                l = lv_v[pl.ds(r + FB, 16)][0]
                for k in range(16):
                    plsc.addupdate(acc.at[l, pl.ds(k * 16, 16)],
                                   bufr1[r, pl.ds(k * 16, 16)])

            return carry

        lax.fori_loop(0, FB, acc_row1, 0)

    def flush_check(cc):
        @pl.when(cc >= CHUNK)
        def _flush():
            accum_batch(jnp.int32(CHUNK))
            p1 = pos_v[pl.ds(CHUNK, 16)]
            l1 = lv_v[pl.ds(CHUNK, 16)]
            p2 = pos_v[pl.ds(CHUNK + 16, 16)]
            l2 = lv_v[pl.ds(CHUNK + 16, 16)]
            pos_v[pl.ds(0, 16)] = p1
            lv_v[pl.ds(0, 16)] = l1
            pos_v[pl.ds(16, 16)] = p2
            lv_v[pl.ds(16, 16)] = l2

        return jnp.where(cc >= CHUNK, cc - CHUNK, cc)

    def blk(b, cnt):
        pltpu.sync_copy(sdst.at[pl.ds(b * IBLK, IBLK)], idxb)

        def grp4(g4, cnt2):
            goff = g4 * 64
            lvs = []
            packs = []
            for r in range(4):
                iv = idxb[pl.ds(goff + r * 16, 16)]
                lv = iv - w * R
                m = (lv >= 0) & (lv < R)
                lvs.append(lv)
                packs.append(tree_add(
                    jnp.where(m, 65536 + lv * 16 + iota16, 0)))
            cc4 = cnt2
            for r in range(4):
                packed = packs[r]
                lv = lvs[r]
                nm = packed >> 16
                base = b * IBLK + goff + r * 16
                cc = cc4

                @pl.when(nm == 1)
                def _one(packed=packed, base=base, cc=cc):
                    enc = packed & 65535
                    lane = enc & 15
                    lval = enc >> 4
                    pos_v[pl.ds(cc, 16)] = jnp.full((16,), base + lane,
                                                    jnp.int32)
                    lv_v[pl.ds(cc, 16)] = jnp.full((16,), lval, jnp.int32)

                @pl.when(nm > 1)
                def _multi(lv=lv, base=base, cc=cc):
                    lvtmp[pl.ds(0, 16)] = lv

                    def ml(lane, ccl):
                        lvl = lvtmp[pl.ds(lane, 16)][0]
                        cond = (lvl >= 0) & (lvl < R)

                        @pl.when(cond)
                        def _st():
                            pos_v[pl.ds(ccl, 16)] = jnp.full(
                                (16,), base + lane, jnp.int32)
                            lv_v[pl.ds(ccl, 16)] = jnp.full((16,), lvl,
                                                            jnp.int32)

                        return ccl + jnp.where(cond, 1, 0)

                    lax.fori_loop(0, 16, ml, cc)

                cc4 = cc4 + nm
                cc4 = flush_check(cc4)
            return cc4

        return lax.fori_loop(0, IBLK // 64, grp4, cnt)

    cnt_end = lax.fori_loop(0, NIB, blk, jnp.int32(0))

    @pl.when(cnt_end > 0)
    def _tail():
        accum_batch(cnt_end)

    pltpu.sync_copy(acc, out.at[pl.ds(w * R, R)])


@functools.lru_cache(maxsize=None)
def _build_sc_kernels():
    mesh = plsc.VectorSubcoreMesh(core_axis_name="c", subcore_axis_name="s",
                                  num_cores=NC, num_subcores=NS)
    gather = pl.kernel(
        _sc_gather_body,
        out_type=(jax.ShapeDtypeStruct((E_PAD, DT), jnp.float32),
                  jax.ShapeDtypeStruct((E_PAD, DT), jnp.float32)),
        mesh=mesh,
        scratch_types=[pltpu.VMEM((CPW, CG), jnp.int32),
                       pltpu.VMEM((CPW, CG), jnp.int32),
                       pltpu.VMEM((CG, DT), jnp.float32),
                       pltpu.VMEM((CG, DT), jnp.float32),
                       pltpu.VMEM((CG, DT), jnp.float32),
                       pltpu.VMEM((CG, DT), jnp.float32),
                       pltpu.SemaphoreType.DMA,
                       pltpu.SemaphoreType.DMA,
                       pltpu.SemaphoreType.DMA,
                       pltpu.SemaphoreType.DMA])
    scatter = pl.kernel(
        _sc_scatter_body,
        out_type=jax.ShapeDtypeStruct((N_ACC, DT), jnp.float32),
        mesh=mesh,
        scratch_types=[pltpu.VMEM((IBLK,), jnp.int32),
                       pltpu.VMEM((176,), jnp.int32),
                       pltpu.VMEM((176,), jnp.int32),
                       pltpu.VMEM((32,), jnp.int32),
                       pltpu.VMEM((FB, DT), jnp.float32),
                       pltpu.VMEM((FB, DT), jnp.float32),
                       pltpu.VMEM((R, DT), jnp.float32),
                       pltpu.SemaphoreType.DMA,
                       pltpu.SemaphoreType.DMA])
    return gather, scatter


def _sc_gather(dtab, stab, gdst3, gsrc3):
    return _build_sc_kernels()[0](dtab, stab, gdst3, gsrc3)


def _sc_scatter(contrib, sdst):
    return _build_sc_kernels()[1](contrib, sdst)


def _final_body(p0, h, Wo_a, Wo_h, bo, out):
    a = p0[...]
    den0 = a[:, H:H + 1] + 1e-16
    den1 = a[:, H + 1:H + 2] + 1e-16
    agg = jnp.concatenate([a[:, :DH] / den0, a[:, DH:H] / den1], axis=1)
    out[...] = agg @ Wo_a[...] + h[...] @ Wo_h[...] + bo[...]


def _final_call(p0, h, Wo_a, Wo_h, bo_r):
    blk = lambda w: pl.BlockSpec((NBLK, w), lambda i: (i, 0))
    return pl.pallas_call(
        _final_body,
        grid=(NB_N,),
        in_specs=[blk(DT), blk(H),
                  _full(Wo_a.shape), _full(Wo_h.shape), _full(bo_r.shape)],
        out_specs=blk(H),
        out_shape=jax.ShapeDtypeStruct((N, H), jnp.float32),
    )(p0, h, Wo_a, Wo_h, bo_r)


def kernel(x, memory, mem_ts, mailbox, mail_ts, edge_ts, edge_feat, h_hist,
           hist_ts, W_t, b_t, W_te, b_te, Wi, Wh, bi, bh, W_feat, b_feat,
           Wq, Wk, Wv, Wo, bo, W_ct, b_ct, Wc1, bc1, Wc2, bc2,
           edge_index, is_remote):
    f32 = jnp.float32
    row = lambda w: w[None, :]
    h, dtab, stab = _node_call(
        x, memory, mem_ts[:, None], mailbox, mail_ts[:, None], h_hist,
        hist_ts[:, None], is_remote[:, None].astype(f32),
        row(W_t), row(b_t), Wi[:H], Wi[H:], Wh, row(bi), row(bh),
        W_feat, row(b_feat), row(W_ct), row(b_ct), Wc1[:H], Wc1[H:],
        row(bc1), Wc2, row(bc2), Wq, Wk[:H], Wv[:H])

    src = edge_index[0]
    dst = edge_index[1]
    pad = E_PAD - E
    gdst3 = jnp.pad(dst, (0, pad)).reshape(NW, CPW, CG)
    gsrc3 = jnp.pad(src, (0, pad)).reshape(NW, CPW, CG)
    sdst = jnp.pad(dst, (0, pad), constant_values=N)
    ets_pad = jnp.pad(edge_ts, (0, pad))[:, None]
    ef_pad = jnp.pad(edge_feat, ((0, pad), (0, 0)))

    qdt, kvt = _sc_gather(dtab, stab, gdst3, gsrc3)
    W_edge = jnp.concatenate([Wk[H:], Wv[H:]], axis=1)
    contrib = _edge_call(qdt, kvt, ets_pad, ef_pad, row(W_te), row(b_te), W_edge)
    acc = _sc_scatter(contrib, sdst)
    return _final_call(acc, h, Wo[:H], Wo[H:], row(bo))
